# baseline (speedup 1.0000x reference)
import functools

import jax
import jax.numpy as jnp
from jax import lax
from jax.experimental import pallas as pl
from jax.experimental.pallas import tpu as pltpu
from jax.experimental.pallas import tpu_sc as plsc

NC = 2
NS = 16
NW = NC * NS
G = 128
CH = 2 * G
SLAB_CH = 2


@functools.cache
def _make_gather(B: int, L: int, V: int, D: int):
    N = B * L
    n_ch = N // CH
    assert N % (NW * CH * SLAB_CH) == 0
    ch_per_w = n_ch // NW
    n_s = ch_per_w // SLAB_CH
    assert n_s % 2 == 0
    slab_rows = SLAB_CH * G
    mesh = plsc.VectorSubcoreMesh(core_axis_name="c", subcore_axis_name="s")

    @functools.partial(
        pl.kernel,
        mesh=mesh,
        out_type=jax.ShapeDtypeStruct((N // 2, 2 * D), jnp.float32),
        compiler_params=pltpu.CompilerParams(use_tc_tiling_on_sc=False),
        scratch_types=[
            pltpu.VMEM((ch_per_w, 2, G), jnp.int32),
            pltpu.VMEM((2, 2, slab_rows, D), jnp.float32),
            pltpu.SemaphoreType.DMA,
            pltpu.SemaphoreType.DMA,
            pltpu.SemaphoreType.DMA,
            pltpu.SemaphoreType.DMA,
        ],
    )
    def gather_kernel(table_hbm, idx_hbm, out_hbm, idx_v, slab_v,
                      gsem0, gsem1, osem0, osem1):
        wid = lax.axis_index("s") * NC + lax.axis_index("c")
        ch0 = wid * ch_per_w
        row0 = ch0 * G

        pltpu.sync_copy(idx_hbm.at[pl.ds(ch0, ch_per_w)], idx_v)

        gsems = (gsem0, gsem1)
        osems = (osem0, osem1)

        def fill(s, p):
            for j in range(SLAB_CH):
                for h in range(2):
                    pltpu.async_copy(
                        table_hbm.at[idx_v.at[s * SLAB_CH + j, h]],
                        slab_v.at[p, h, pl.ds(j * G, G)],
                        gsems[p],
                    )

        def drain(s, p):
            for j in range(SLAB_CH):
                for h in range(2):
                    pltpu.make_async_copy(
                        table_hbm.at[idx_v.at[s * SLAB_CH + j, h]],
                        slab_v.at[p, h, pl.ds(j * G, G)],
                        gsems[p],
                    ).wait()

        def out_slice(s, h):
            return out_hbm.at[pl.ds(row0 + s * slab_rows, slab_rows),
                              pl.ds(h * D, D)]

        def write(s, p):
            for h in range(2):
                pltpu.async_copy(slab_v.at[p, h], out_slice(s, h), osems[p])

        def write_wait(s, p):
            for h in range(2):
                pltpu.make_async_copy(
                    slab_v.at[p, h], out_slice(s, h), osems[p]
                ).wait()

        fill(0, 0)

        def body(t, _):
            for p in range(2):
                s = t * 2 + p
                q = 1 - p

                @pl.when(s + 1 < n_s)
                def _():
                    @pl.when(s >= 1)
                    def _():
                        write_wait(s - 1, q)
                    fill(s + 1, q)

                drain(s, p)
                write(s, p)
            return 0

        lax.fori_loop(0, n_s // 2, body, 0)

        write_wait(n_s - 2, 0)
        write_wait(n_s - 1, 1)

    return gather_kernel


@functools.cache
def _make_relayout(B: int, L: int, D: int, BB: int):
    rows_per_blk = BB * L // 2

    def body(x_ref, o_ref):
        x = x_ref[...]
        left = x[:, :D]
        right = x[:, D:]
        y = jnp.concatenate([left[:, None, :], right[:, None, :]], axis=1)
        o_ref[...] = y.reshape(BB, L, D)

    return pl.pallas_call(
        body,
        grid=(B // BB,),
        in_specs=[pl.BlockSpec((rows_per_blk, 2 * D), lambda i: (i, 0))],
        out_specs=pl.BlockSpec((BB, L, D), lambda i: (i, 0, 0)),
        out_shape=jax.ShapeDtypeStruct((B, L, D), jnp.float32),
    )


def kernel(seqs, att_mask, word_embedding):
    B, L = seqs.shape
    V, D = word_embedding.shape
    N = B * L
    idx_f = (seqs.astype(jnp.int32)
             .reshape(N // CH, G, 2)
             .transpose(0, 2, 1))
    packed = _make_gather(B, L, V, D)(word_embedding, idx_f)
    return _make_relayout(B, L, D, 8)(packed)


# speedup vs baseline: 1.4052x; 1.4052x over previous
import functools

import jax
import jax.numpy as jnp
from jax import lax
from jax.experimental import pallas as pl
from jax.experimental.pallas import tpu as pltpu
from jax.experimental.pallas import tpu_sc as plsc

NC = 2
NS = 16
NW = NC * NS
G = 128
CH = 2 * G
SLAB_CH = 2


@functools.cache
def _make_gather(B: int, L: int, V: int, D: int):
    N = B * L
    n_ch = N // CH
    assert N % (NW * CH * SLAB_CH) == 0
    ch_per_w = n_ch // NW
    n_s = ch_per_w // SLAB_CH
    assert n_s % 2 == 0
    slab_rows = SLAB_CH * G
    mesh = plsc.VectorSubcoreMesh(core_axis_name="c", subcore_axis_name="s")

    @functools.partial(
        pl.kernel,
        mesh=mesh,
        out_type=jax.ShapeDtypeStruct((N // 2, 2 * D), jnp.float32),
        compiler_params=pltpu.CompilerParams(use_tc_tiling_on_sc=False),
        scratch_types=[
            pltpu.VMEM((2, ch_per_w, G), jnp.int32),
            pltpu.VMEM((2, 2, slab_rows, D), jnp.float32),
            pltpu.SemaphoreType.DMA,
            pltpu.SemaphoreType.DMA,
            pltpu.SemaphoreType.DMA,
            pltpu.SemaphoreType.DMA,
        ],
    )
    def gather_kernel(table_hbm, idx_hbm, out_hbm, idx_v, slab_v,
                      gsem0, gsem1, osem0, osem1):
        wid = lax.axis_index("s") * NC + lax.axis_index("c")
        ch0 = wid * ch_per_w
        row0 = ch0 * G

        pltpu.sync_copy(idx_hbm.at[:, pl.ds(ch0, ch_per_w)], idx_v)

        gsems = (gsem0, gsem1)
        osems = (osem0, osem1)

        def fill(s, p):
            for j in range(SLAB_CH):
                for h in range(2):
                    pltpu.async_copy(
                        table_hbm.at[idx_v.at[h, s * SLAB_CH + j]],
                        slab_v.at[p, h, pl.ds(j * G, G)],
                        gsems[p],
                    )

        def drain(s, p):
            for j in range(SLAB_CH):
                for h in range(2):
                    pltpu.make_async_copy(
                        table_hbm.at[idx_v.at[h, s * SLAB_CH + j]],
                        slab_v.at[p, h, pl.ds(j * G, G)],
                        gsems[p],
                    ).wait()

        def out_slice(s, h):
            return out_hbm.at[pl.ds(row0 + s * slab_rows, slab_rows),
                              pl.ds(h * D, D)]

        def write(s, p):
            for h in range(2):
                pltpu.async_copy(slab_v.at[p, h], out_slice(s, h), osems[p])

        def write_wait(s, p):
            for h in range(2):
                pltpu.make_async_copy(
                    slab_v.at[p, h], out_slice(s, h), osems[p]
                ).wait()

        fill(0, 0)

        def body(t, _):
            for p in range(2):
                s = t * 2 + p
                q = 1 - p

                @pl.when(s + 1 < n_s)
                def _():
                    @pl.when(s >= 1)
                    def _():
                        write_wait(s - 1, q)
                    fill(s + 1, q)

                drain(s, p)
                write(s, p)
            return 0

        lax.fori_loop(0, n_s // 2, body, 0)

        write_wait(n_s - 2, 0)
        write_wait(n_s - 1, 1)

    return gather_kernel


def kernel(seqs, att_mask, word_embedding):
    B, L = seqs.shape
    V, D = word_embedding.shape
    N = B * L
    idx_f = (seqs.astype(jnp.int32)
             .reshape(N // CH, G, 2)
             .transpose(2, 0, 1))
    packed = _make_gather(B, L, V, D)(word_embedding, idx_f)
    return packed.reshape(B, L, D)


# speedup vs baseline: 3.1229x; 2.2225x over previous
import functools

import jax
import jax.numpy as jnp
from jax import lax
from jax.experimental import pallas as pl
from jax.experimental.pallas import tpu as pltpu
from jax.experimental.pallas import tpu_sc as plsc

NC = 2
NS = 16
NW = NC * NS
R = 4
SPLITS = ((0, 128), (128, 72))


@functools.cache
def _make_gather(B: int, L: int, V: int, D: int):
    assert B % (NW * R) == 0
    assert sum(g for _, g in SPLITS) == L
    b_per_w = B // NW
    n_s = b_per_w // R
    assert n_s % 2 == 0
    mesh = plsc.VectorSubcoreMesh(core_axis_name="c", subcore_axis_name="s")

    @functools.partial(
        pl.kernel,
        mesh=mesh,
        out_type=jax.ShapeDtypeStruct((B, L, 2 * D), jnp.float32),
        compiler_params=pltpu.CompilerParams(use_tc_tiling_on_sc=False),
        scratch_types=[
            pltpu.VMEM((b_per_w, L), jnp.int32),
            pltpu.VMEM((2, R, L, D), jnp.float32),
            pltpu.SemaphoreType.DMA,
            pltpu.SemaphoreType.DMA,
            pltpu.SemaphoreType.DMA,
            pltpu.SemaphoreType.DMA,
        ],
    )
    def gather_kernel(table_hbm, seqs_hbm, out_hbm, idx_v, slab_v,
                      gsem0, gsem1, osem0, osem1):
        wid = lax.axis_index("s") * NC + lax.axis_index("c")
        b0 = wid * b_per_w

        pltpu.sync_copy(seqs_hbm.at[pl.ds(b0, b_per_w)], idx_v)

        gsems = (gsem0, gsem1)
        osems = (osem0, osem1)

        def fill(s, p):
            for r in range(R):
                for o, g in SPLITS:
                    pltpu.async_copy(
                        table_hbm.at[idx_v.at[s * R + r, pl.ds(o, g)]],
                        slab_v.at[p, r, pl.ds(o, g)],
                        gsems[p],
                    )

        def drain(s, p):
            for r in range(R):
                for o, g in SPLITS:
                    pltpu.make_async_copy(
                        table_hbm.at[idx_v.at[s * R + r, pl.ds(o, g)]],
                        slab_v.at[p, r, pl.ds(o, g)],
                        gsems[p],
                    ).wait()

        def out_slice(s):
            return out_hbm.at[pl.ds(b0 + s * R, R), :, pl.ds(0, D)]

        fill(0, 0)

        def body(t, _):
            for p in range(2):
                s = t * 2 + p
                q = 1 - p

                @pl.when(s + 1 < n_s)
                def _():
                    @pl.when(s >= 1)
                    def _():
                        pltpu.make_async_copy(
                            slab_v.at[q], out_slice(s - 1), osems[q]
                        ).wait()
                    fill(s + 1, q)

                drain(s, p)
                pltpu.async_copy(slab_v.at[p], out_slice(s), osems[p])
            return 0

        lax.fori_loop(0, n_s // 2, body, 0)

        pltpu.make_async_copy(slab_v.at[0], out_slice(n_s - 2), osems[0]).wait()
        pltpu.make_async_copy(slab_v.at[1], out_slice(n_s - 1), osems[1]).wait()

    return gather_kernel


def kernel(seqs, att_mask, word_embedding):
    B, L = seqs.shape
    V, D = word_embedding.shape
    padded = _make_gather(B, L, V, D)(word_embedding, seqs.astype(jnp.int32))
    return padded[:, :, :D]
